# baseline (speedup 1.0000x reference)
import functools

import jax
import jax.numpy as jnp
from jax.experimental import pallas as pl


def _small_gemm_kernel(x_ref, w_ref, o_ref):
    o_ref[...] = jnp.dot(x_ref[...], w_ref[...],
                         preferred_element_type=jnp.float32)


def _layer1_kernel(adj_ref, s_ref, b_ref, w2_ref, o_ref):
    t = jnp.dot(adj_ref[...], s_ref[...],
                preferred_element_type=jnp.float32) + b_ref[...]
    h = jnp.where(t > 0, t, 0.2 * t)
    o_ref[...] = jnp.dot(h, w2_ref[...], preferred_element_type=jnp.float32)


def _layer2_kernel(adj_ref, s_ref, b_ref, o_ref):
    o_ref[...] = jnp.dot(adj_ref[...], s_ref[...],
                         preferred_element_type=jnp.float32) + b_ref[...]


def _pick_block(n):
    for bm in (400, 200, 80, 40, 8):
        if n % bm == 0:
            return bm
    return n


@jax.jit
def kernel(x, adj, W1, b1, W2, b2):
    n, d_in = x.shape
    d_hid = W1.shape[1]
    d_out = W2.shape[1]
    bm = _pick_block(n)
    grid = (n // bm,)

    s1 = pl.pallas_call(
        _small_gemm_kernel,
        out_shape=jax.ShapeDtypeStruct((n, d_hid), jnp.float32),
    )(x, W1)

    b1r = b1.reshape(1, d_hid)
    b2r = b2.reshape(1, d_out)

    s2 = pl.pallas_call(
        _layer1_kernel,
        grid=grid,
        in_specs=[
            pl.BlockSpec((bm, n), lambda i: (i, 0)),
            pl.BlockSpec((n, d_hid), lambda i: (0, 0)),
            pl.BlockSpec((1, d_hid), lambda i: (0, 0)),
            pl.BlockSpec((d_hid, d_out), lambda i: (0, 0)),
        ],
        out_specs=pl.BlockSpec((bm, d_out), lambda i: (i, 0)),
        out_shape=jax.ShapeDtypeStruct((n, d_out), jnp.float32),
    )(adj, s1, b1r, W2)

    out = pl.pallas_call(
        _layer2_kernel,
        grid=grid,
        in_specs=[
            pl.BlockSpec((bm, n), lambda i: (i, 0)),
            pl.BlockSpec((n, d_out), lambda i: (0, 0)),
            pl.BlockSpec((1, d_out), lambda i: (0, 0)),
        ],
        out_specs=pl.BlockSpec((bm, d_out), lambda i: (i, 0)),
        out_shape=jax.ShapeDtypeStruct((n, d_out), jnp.float32),
    )(adj, s2, b2r)

    return out


# speedup vs baseline: 1.0501x; 1.0501x over previous
import jax
import jax.numpy as jnp
from jax.experimental import pallas as pl
from jax.experimental.pallas import tpu as pltpu


def _make_kernel(steps, bm):
    def _gcn_kernel(x_ref, adj_ref, w1_ref, b1_ref, w2_ref, b2_ref,
                    out_ref, s1_ref, s2_ref):
        i = pl.program_id(0)

        @pl.when(i == 0)
        def _():
            s1_ref[...] = jnp.dot(x_ref[...], w1_ref[...],
                                  preferred_element_type=jnp.float32)

        @pl.when(i < steps)
        def _():
            t = jnp.dot(adj_ref[...], s1_ref[...],
                        preferred_element_type=jnp.float32) + b1_ref[...]
            h = jnp.where(t > 0, t, 0.2 * t)
            s2_ref[pl.ds(i * bm, bm), :] = jnp.dot(
                h, w2_ref[...], preferred_element_type=jnp.float32)

        @pl.when(i >= steps)
        def _():
            out_ref[...] = jnp.dot(adj_ref[...], s2_ref[...],
                                   preferred_element_type=jnp.float32) + b2_ref[...]

    return _gcn_kernel


def _pick_block(n):
    for bm in (400, 200, 80, 40, 8):
        if n % bm == 0:
            return bm
    return n


@jax.jit
def kernel(x, adj, W1, b1, W2, b2):
    n, d_in = x.shape
    d_hid = W1.shape[1]
    d_out = W2.shape[1]
    bm = _pick_block(n)
    steps = n // bm

    b1r = b1.reshape(1, d_hid)
    b2r = b2.reshape(1, d_out)

    out = pl.pallas_call(
        _make_kernel(steps, bm),
        grid=(2 * steps,),
        in_specs=[
            pl.BlockSpec((n, d_in), lambda i: (0, 0)),
            pl.BlockSpec((bm, n), lambda i: (jax.lax.rem(i, steps), 0)),
            pl.BlockSpec((d_in, d_hid), lambda i: (0, 0)),
            pl.BlockSpec((1, d_hid), lambda i: (0, 0)),
            pl.BlockSpec((d_hid, d_out), lambda i: (0, 0)),
            pl.BlockSpec((1, d_out), lambda i: (0, 0)),
        ],
        out_specs=pl.BlockSpec(
            (bm, d_out),
            lambda i: (jnp.maximum(i - steps, 0), 0)),
        out_shape=jax.ShapeDtypeStruct((n, d_out), jnp.float32),
        scratch_shapes=[
            pltpu.VMEM((n, d_hid), jnp.float32),
            pltpu.VMEM((n, d_out), jnp.float32),
        ],
    )(x, adj, W1, b1r, W2, b2r)

    return out
